# baseline (speedup 1.0000x reference)
import functools

import jax
import jax.numpy as jnp
from jax import lax
from jax.experimental import pallas as pl
from jax.experimental.pallas import tpu as pltpu
from jax.experimental.pallas import tpu_sc as plsc

_B = 16384
_L = 200
_D = 64
_C = 2
_V = 1000000

_NC = 2
_NS = 16
_NW = _NC * _NS
_NB = _B // _NW
_G = _D // 16
_SPLIT = 128
_REST = _L - _SPLIT
_GB = 8
_NGRP = _NB // _GB
_NIB = 4


def _body(table_h, xflat_h, w_h, bp_h, out_h,
          idx_v, rows_v, out_v, w_v, b_v, row_sems, idx_sems):
  wid = lax.axis_index("s") * _NC + lax.axis_index("c")
  base_b = wid * _NB

  pltpu.sync_copy(w_h, w_v)
  pltpu.sync_copy(bp_h, b_v)

  w_regs = [[w_v[c, pl.ds(g * 16, 16)] for g in range(_G)] for c in range(_C)]
  bp_reg = b_v[...]
  lane = lax.iota(jnp.int32, 16)
  zero = jnp.zeros((16,), jnp.float32)
  neg = jnp.full((16,), -jnp.inf, dtype=jnp.float32)

  def stage_idx(grp, q):
    off = pl.multiple_of((base_b + grp * _GB) * _L, 8)
    pltpu.async_copy(xflat_h.at[pl.ds(off, _GB * _L)], idx_v.at[q],
                     idx_sems.at[q])

  def wait_idx(q):
    pltpu.make_async_copy(xflat_h.at[pl.ds(0, _GB * _L)], idx_v.at[q],
                          idx_sems.at[q]).wait()

  def issue(q, j):
    off = j * _L
    pltpu.async_copy(table_h.at[idx_v.at[q, pl.ds(off, _SPLIT)]],
                     rows_v.at[j, pl.ds(0, _SPLIT)], row_sems.at[j])
    pltpu.async_copy(table_h.at[idx_v.at[q, pl.ds(off + _SPLIT, _REST)]],
                     rows_v.at[j, pl.ds(_SPLIT, _REST)], row_sems.at[j])

  def wait_rows(j):
    pltpu.make_async_copy(table_h.at[pl.ds(0, _L)], rows_v.at[j],
                          row_sems.at[j]).wait()

  def compute(j, v):
    def red(r, accs):
      return tuple(
          jnp.maximum(accs[g], rows_v[j, r, pl.ds(g * 16, 16)])
          for g in range(_G))

    accs = lax.fori_loop(0, _L, red, (neg,) * _G, unroll=8)

    for c in range(_C):
      p = accs[0] * w_regs[c][0]
      for g in range(1, _G):
        p = p + accs[g] * w_regs[c][g]
      v = jnp.where(lane == (_C * j + c), jnp.sum(p), v)
    return v

  stage_idx(0, 0)
  stage_idx(1, 1)
  wait_idx(0)
  for j in range(_GB):
    issue(0, j)

  @pl.loop(0, _NGRP, step=_NIB)
  def _(g4):
    for q in range(_NIB):
      g = g4 + q

      @pl.when(g + 1 < _NGRP)
      def _():
        wait_idx((q + 1) % _NIB)

      @pl.when(g + 2 < _NGRP)
      def _():
        stage_idx(g + 2, (q + 2) % _NIB)

      v = zero
      for j in range(_GB):
        wait_rows(j)
        v = compute(j, v)

        @pl.when(g + 1 < _NGRP)
        def _():
          issue((q + 1) % _NIB, j)

      out_v[pl.ds(pl.multiple_of(g * _GB * _C, 16), 16)] = v + bp_reg

  pltpu.sync_copy(out_v, out_h.at[pl.ds(base_b * _C, _NB * _C)])


_TCOLS = _V // 128
_TFULL = _TCOLS // _NW
_TEXTRA = _TCOLS % _NW


def _tr_body(tt_h, tail_h, out_h, colbuf_v, tbuf_v, in_sems, out_sems):
  wid = lax.axis_index("s") * _NC + lax.axis_index("c")
  start = _TFULL * wid + jnp.minimum(wid, _TEXTRA)
  cnt = _TFULL + jnp.where(wid < _TEXTRA, 1, 0)

  lanes = lax.iota(jnp.int32, 16)

  def issue_in(k, s):
    pltpu.async_copy(tt_h.at[:, pl.ds(k * 128, 128)], colbuf_v.at[s],
                     in_sems.at[s])

  def wait_in(s):
    pltpu.make_async_copy(tt_h.at[:, pl.ds(0, 128)], colbuf_v.at[s],
                          in_sems.at[s]).wait()

  def transpose(s, n_pairs):
    def m_body(m2, _):
      for h in range(2):
        col = 2 * m2 + h
        for g in range(_G):
          v = plsc.load_gather(
              colbuf_v.at[s], [g * 16 + lanes, jnp.full((16,), col, jnp.int32)])
          tbuf_v[s, m2, pl.ds(h * 64 + g * 16, 16)] = v
      return 0
    lax.fori_loop(0, n_pairs, m_body, 0, unroll=4)

  def issue_out(k, s):
    pltpu.async_copy(tbuf_v.at[s], out_h.at[pl.ds(k * 64, 64)], out_sems.at[s])

  def wait_out(s):
    pltpu.make_async_copy(tbuf_v.at[s], out_h.at[pl.ds(0, 64)],
                          out_sems.at[s]).wait()

  issue_in(start, 0)

  @pl.loop(0, cnt, step=2)
  def _(i2):
    for s in range(2):
      i = i2 + s

      @pl.when(i < cnt)
      def _():
        k = start + i

        @pl.when(i + 1 < cnt)
        def _():
          issue_in(k + 1, 1 - s)

        wait_in(s)

        @pl.when(i >= 2)
        def _():
          wait_out(s)

        transpose(s, 64)
        issue_out(k, s)

  wait_out(cnt % 2)
  wait_out((cnt + 1) % 2)

  @pl.when(wid == _NW - 1)
  def _():
    pltpu.sync_copy(tail_h, tbuf_v.at[0, pl.ds(0, 32)])
    pltpu.sync_copy(tbuf_v.at[0, pl.ds(0, 32)],
                    out_h.at[pl.ds(_TCOLS * 64, 32)])


def kernel(x, emb_table, W, b):
  xflat = x.reshape(-1).astype(jnp.int32)
  btiled = jnp.tile(b.astype(jnp.float32), 16 // _C)

  transposer = pl.kernel(
      _tr_body,
      out_type=jax.ShapeDtypeStruct((_V // 2, 128), jnp.float32),
      mesh=plsc.VectorSubcoreMesh(core_axis_name="c", subcore_axis_name="s"),
      compiler_params=pltpu.CompilerParams(
          needs_layout_passes=False, use_tc_tiling_on_sc=True),
      scratch_types=[
          pltpu.VMEM((2, _D, 128), jnp.float32),
          pltpu.VMEM((2, 64, 128), jnp.float32),
          pltpu.SemaphoreType.DMA((2,)),
          pltpu.SemaphoreType.DMA((2,)),
      ],
  )
  tail_lin = emb_table[_TCOLS * 128:, :].reshape(32, 128)
  table2 = transposer(emb_table.T, tail_lin).reshape(_V, _D)

  run = pl.kernel(
      _body,
      out_type=jax.ShapeDtypeStruct((_B * _C,), jnp.float32),
      mesh=plsc.VectorSubcoreMesh(core_axis_name="c", subcore_axis_name="s"),
      compiler_params=pltpu.CompilerParams(
          needs_layout_passes=False, use_tc_tiling_on_sc=False),
      scratch_types=[
          pltpu.VMEM((_NIB, _GB * _L), jnp.int32),
          pltpu.VMEM((_GB, _L, _D), jnp.float32),
          pltpu.VMEM((_NB * _C,), jnp.float32),
          pltpu.VMEM((_C, _D), jnp.float32),
          pltpu.VMEM((16,), jnp.float32),
          pltpu.SemaphoreType.DMA((_GB,)),
          pltpu.SemaphoreType.DMA((_NIB,)),
      ],
  )
  return run(table2, xflat, W, btiled).reshape(_B, _C)
